# baseline (speedup 1.0000x reference)
import functools

import jax
import jax.numpy as jnp
from jax import lax
from jax.experimental import pallas as pl
from jax.experimental.pallas import tpu as pltpu
from jax.experimental.pallas import tpu_sc as plsc

_B = 16384
_D = 32
_EPS = 1e-5


def _sc_gather4(user, item, ug_tab, ig_tab, um_tab, im_tab):
    info = plsc.get_sparse_core_info()
    nc, ns = info.num_cores, info.num_subcores
    nw = nc * ns
    bpw = _B // nw

    mesh = plsc.VectorSubcoreMesh(core_axis_name="c", subcore_axis_name="s")

    @functools.partial(
        pl.kernel,
        mesh=mesh,
        out_type=[jax.ShapeDtypeStruct((_B, _D), jnp.float32)] * 4,
        scratch_types=[
            pltpu.VMEM((bpw,), jnp.int32),
            pltpu.VMEM((bpw,), jnp.int32),
            pltpu.VMEM((bpw, _D), jnp.float32),
            pltpu.VMEM((bpw, _D), jnp.float32),
            pltpu.VMEM((bpw, _D), jnp.float32),
            pltpu.VMEM((bpw, _D), jnp.float32),
            pltpu.SemaphoreType.DMA,
        ],
        compiler_params=pltpu.CompilerParams(use_tc_tiling_on_sc=False),
    )
    def gather_kernel(user_hbm, item_hbm, ug_hbm, ig_hbm, um_hbm, im_hbm,
                      out_ug, out_ig, out_um, out_im,
                      uidx, iidx, r0, r1, r2, r3, sem):
        wid = lax.axis_index("s") * nc + lax.axis_index("c")
        base = wid * bpw
        pltpu.sync_copy(user_hbm.at[pl.ds(base, bpw)], uidx)
        pltpu.sync_copy(item_hbm.at[pl.ds(base, bpw)], iidx)
        c0 = pltpu.async_copy(ug_hbm.at[uidx], r0, sem)
        c1 = pltpu.async_copy(ig_hbm.at[iidx], r1, sem)
        c2 = pltpu.async_copy(um_hbm.at[uidx], r2, sem)
        c3 = pltpu.async_copy(im_hbm.at[iidx], r3, sem)
        c0.wait()
        c1.wait()
        c2.wait()
        c3.wait()
        pltpu.sync_copy(r0, out_ug.at[pl.ds(base, bpw)])
        pltpu.sync_copy(r1, out_ig.at[pl.ds(base, bpw)])
        pltpu.sync_copy(r2, out_um.at[pl.ds(base, bpw)])
        pltpu.sync_copy(r3, out_im.at[pl.ds(base, bpw)])

    return gather_kernel(user, item, ug_tab, ig_tab, um_tab, im_tab)


def _tc_body(ug, ig, um, im,
             w0, b0, g0, be0, w1, b1, g1, be1, w2, b2, g2, be2,
             wo_g, wo_m, bo, out):
    x = jnp.concatenate([um[...], im[...]], axis=1)
    for w, b, g, be in ((w0, b0, g0, be0), (w1, b1, g1, be1), (w2, b2, g2, be2)):
        x = jnp.dot(x, w[...], preferred_element_type=jnp.float32) + b[...]
        mean = jnp.mean(x, axis=0, keepdims=True)
        var = jnp.mean((x - mean) ** 2, axis=0, keepdims=True)
        x = (x - mean) * lax.rsqrt(var + _EPS) * g[...] + be[...]
        x = jnp.maximum(x, 0.0)
    gmf = ug[...] * ig[...]
    s = (jnp.dot(gmf, wo_g[...], preferred_element_type=jnp.float32)
         + jnp.dot(x, wo_m[...], preferred_element_type=jnp.float32)
         + bo[...])
    out[...] = 1.0 / (1.0 + jnp.exp(-s))


def _tc_forward(ug, ig, um, im, params):
    out = pl.pallas_call(
        _tc_body,
        out_shape=jax.ShapeDtypeStruct((_B, 1), jnp.float32),
    )(ug, ig, um, im, *params)
    return out


def kernel(user, item, user_gmf_tab, item_gmf_tab, user_mlp_tab, item_mlp_tab,
           W0, b0, g0, be0, W1, b1, g1, be1, W2, b2, g2, be2, Wo, bo):
    user = user.astype(jnp.int32)
    item = item.astype(jnp.int32)
    ug, ig, um, im = _sc_gather4(user, item, user_gmf_tab, item_gmf_tab,
                                 user_mlp_tab, item_mlp_tab)
    params = (
        W0, b0.reshape(1, -1), g0.reshape(1, -1), be0.reshape(1, -1),
        W1, b1.reshape(1, -1), g1.reshape(1, -1), be1.reshape(1, -1),
        W2, b2.reshape(1, -1), g2.reshape(1, -1), be2.reshape(1, -1),
        Wo[:_D], Wo[_D:], bo.reshape(1, 1),
    )
    out = _tc_forward(ug, ig, um, im, params)
    return jnp.squeeze(out, axis=-1)


# speedup vs baseline: 1.0008x; 1.0008x over previous
import functools

import jax
import jax.numpy as jnp
from jax import lax
from jax.experimental import pallas as pl
from jax.experimental.pallas import tpu as pltpu
from jax.experimental.pallas import tpu_sc as plsc

_B = 16384
_D = 32
_EPS = 1e-5


def _sc_gather4(user, item, ug_tab, ig_tab, um_tab, im_tab):
    info = plsc.get_sparse_core_info()
    nc, ns = info.num_cores, info.num_subcores
    nw = nc * ns
    bpw = _B // nw

    mesh = plsc.VectorSubcoreMesh(core_axis_name="c", subcore_axis_name="s")

    @functools.partial(
        pl.kernel,
        mesh=mesh,
        out_type=[jax.ShapeDtypeStruct((_B, _D), jnp.float32)] * 4,
        scratch_types=[
            pltpu.VMEM((bpw,), jnp.int32),
            pltpu.VMEM((bpw,), jnp.int32),
            pltpu.VMEM((bpw, _D), jnp.float32),
            pltpu.VMEM((bpw, _D), jnp.float32),
            pltpu.VMEM((bpw, _D), jnp.float32),
            pltpu.VMEM((bpw, _D), jnp.float32),
            pltpu.SemaphoreType.DMA,
        ],
        compiler_params=pltpu.CompilerParams(
            use_tc_tiling_on_sc=False, needs_layout_passes=True),
    )
    def gather_kernel(user_hbm, item_hbm, ug_hbm, ig_hbm, um_hbm, im_hbm,
                      out_ug, out_ig, out_um, out_im,
                      uidx, iidx, r0, r1, r2, r3, sem):
        wid = lax.axis_index("s") * nc + lax.axis_index("c")
        base = wid * bpw
        pltpu.sync_copy(user_hbm.at[pl.ds(base, bpw)], uidx)
        pltpu.sync_copy(item_hbm.at[pl.ds(base, bpw)], iidx)
        c0 = pltpu.async_copy(ug_hbm.at[uidx], r0, sem)
        c1 = pltpu.async_copy(ig_hbm.at[iidx], r1, sem)
        c2 = pltpu.async_copy(um_hbm.at[uidx], r2, sem)
        c3 = pltpu.async_copy(im_hbm.at[iidx], r3, sem)
        c0.wait()
        c1.wait()
        c2.wait()
        c3.wait()
        pltpu.sync_copy(r0, out_ug.at[pl.ds(base, bpw)])
        pltpu.sync_copy(r1, out_ig.at[pl.ds(base, bpw)])
        pltpu.sync_copy(r2, out_um.at[pl.ds(base, bpw)])
        pltpu.sync_copy(r3, out_im.at[pl.ds(base, bpw)])

    return gather_kernel(user, item, ug_tab, ig_tab, um_tab, im_tab)


def _tc_body(ug, ig, um, im,
             w0, b0, g0, be0, w1, b1, g1, be1, w2, b2, g2, be2,
             wo_g, wo_m, bo, out):
    x = jnp.concatenate([um[...], im[...]], axis=1)
    for w, b, g, be in ((w0, b0, g0, be0), (w1, b1, g1, be1), (w2, b2, g2, be2)):
        x = jnp.dot(x, w[...], preferred_element_type=jnp.float32) + b[...]
        mean = jnp.mean(x, axis=0, keepdims=True)
        var = jnp.mean((x - mean) ** 2, axis=0, keepdims=True)
        x = (x - mean) * lax.rsqrt(var + _EPS) * g[...] + be[...]
        x = jnp.maximum(x, 0.0)
    gmf = ug[...] * ig[...]
    s = (jnp.dot(gmf, wo_g[...], preferred_element_type=jnp.float32)
         + jnp.dot(x, wo_m[...], preferred_element_type=jnp.float32)
         + bo[...])
    out[...] = 1.0 / (1.0 + jnp.exp(-s))


def _tc_forward(ug, ig, um, im, params):
    out = pl.pallas_call(
        _tc_body,
        out_shape=jax.ShapeDtypeStruct((_B, 1), jnp.float32),
    )(ug, ig, um, im, *params)
    return out


def kernel(user, item, user_gmf_tab, item_gmf_tab, user_mlp_tab, item_mlp_tab,
           W0, b0, g0, be0, W1, b1, g1, be1, W2, b2, g2, be2, Wo, bo):
    user = user.astype(jnp.int32)
    item = item.astype(jnp.int32)
    ug, ig, um, im = _sc_gather4(user, item, user_gmf_tab, item_gmf_tab,
                                 user_mlp_tab, item_mlp_tab)
    params = (
        W0, b0.reshape(1, -1), g0.reshape(1, -1), be0.reshape(1, -1),
        W1, b1.reshape(1, -1), g1.reshape(1, -1), be1.reshape(1, -1),
        W2, b2.reshape(1, -1), g2.reshape(1, -1), be2.reshape(1, -1),
        Wo[:_D], Wo[_D:], bo.reshape(1, 1),
    )
    out = _tc_forward(ug, ig, um, im, params)
    return jnp.squeeze(out, axis=-1)
